# Initial kernel scaffold; baseline (speedup 1.0000x reference)
import functools

import jax
import jax.numpy as jnp
from jax import lax
from jax.experimental import pallas as pl
from jax.experimental.pallas import tpu as pltpu
from jax.experimental.pallas import tpu_sc as plsc

N = 10000
E = 320000
D = 128
EPS = 1e-15

NC = 2
NS = 16
NW = NC * NS
EW = E // NW
C = 80
NCH = EW // C
RPT = N // NS

_mesh = plsc.VectorSubcoreMesh(core_axis_name="c", subcore_axis_name="s")


def _segsum_body(h_hbm, src_hbm, dst_hbm, zeros_hbm, out_hbm,
                 sidx_v, didx_v, rows_v, acc_sh, sem):
    c = lax.axis_index("c")
    s = lax.axis_index("s")
    wid = s * NC + c
    base = s * RPT
    pltpu.sync_copy(zeros_hbm.at[pl.ds(base, RPT)], acc_sh.at[pl.ds(base, RPT)])
    plsc.subcore_barrier()
    pltpu.sync_copy(src_hbm.at[wid], sidx_v)
    pltpu.sync_copy(dst_hbm.at[wid], didx_v)

    def chunk(j, carry):
        pltpu.async_copy(h_hbm.at[sidx_v.at[j]], rows_v, sem).wait()
        pltpu.sync_copy(rows_v, acc_sh.at[didx_v.at[j]], add=True)
        return carry

    lax.fori_loop(0, NCH, chunk, 0)
    plsc.subcore_barrier()
    pltpu.sync_copy(acc_sh.at[pl.ds(base, RPT)], out_hbm.at[c, pl.ds(base, RPT)])


_segsum = pl.kernel(
    _segsum_body,
    out_type=jax.ShapeDtypeStruct((NC, N, D), jnp.float32),
    mesh=_mesh,
    scratch_types=[
        pltpu.VMEM((NCH, C), jnp.int32),
        pltpu.VMEM((NCH, C), jnp.int32),
        pltpu.VMEM((C, D), jnp.float32),
        pltpu.VMEM_SHARED((N, D), jnp.float32),
        pltpu.SemaphoreType.DMA,
    ],
)


def _edge_dots_body(z_hbm, eidx_hbm, out_hbm,
                    sidx_v, didx_v, zs_v, zd_v, vals_v, sem_s, sem_d):
    c = lax.axis_index("c")
    s = lax.axis_index("s")
    wid = s * NC + c
    lane = lax.iota(jnp.int32, 16)

    for p in range(2):
        pltpu.sync_copy(eidx_hbm.at[p, 0, wid], sidx_v)
        pltpu.sync_copy(eidx_hbm.at[p, 1, wid], didx_v)

        def chunk(j, carry):
            cs = pltpu.async_copy(z_hbm.at[sidx_v.at[j]], zs_v, sem_s)
            cd = pltpu.async_copy(z_hbm.at[didx_v.at[j]], zd_v, sem_d)
            cs.wait()
            cd.wait()
            for g in range(C // 16):
                rows = lane + (g * 16)

                def feat(f, acc):
                    col = jnp.full((16,), f, dtype=jnp.int32)
                    sv = plsc.load_gather(zs_v, [rows, col])
                    dv = plsc.load_gather(zd_v, [rows, col])
                    return acc + sv * dv

                acc = lax.fori_loop(0, D, feat, jnp.zeros((16,), jnp.float32),
                                    unroll=8)
                vals_v[j, pl.ds(g * 16, 16)] = acc
            return carry

        lax.fori_loop(0, NCH, chunk, 0)
        pltpu.sync_copy(vals_v, out_hbm.at[p, wid])


_edge_dots = pl.kernel(
    _edge_dots_body,
    out_type=jax.ShapeDtypeStruct((2, NW, NCH, C), jnp.float32),
    mesh=_mesh,
    scratch_types=[
        pltpu.VMEM((NCH, C), jnp.int32),
        pltpu.VMEM((NCH, C), jnp.int32),
        pltpu.VMEM((C, D), jnp.float32),
        pltpu.VMEM((C, D), jnp.float32),
        pltpu.VMEM((NCH, C), jnp.float32),
        pltpu.SemaphoreType.DMA,
        pltpu.SemaphoreType.DMA,
    ],
)


def _layer_body(h_ref, m_ref, w_ref, b_ref, o_ref):
    hm = h_ref[...] + m_ref[0] + m_ref[1]
    y = jnp.dot(hm, w_ref[...], preferred_element_type=jnp.float32,
                precision=lax.Precision.HIGHEST)
    o_ref[...] = jnp.maximum(y + b_ref[...], 0.0)


_layer_tc = pl.pallas_call(
    _layer_body,
    out_shape=jax.ShapeDtypeStruct((N, D), jnp.float32),
)


def _head_body(h_ref, wmu_ref, bmu_ref, wlv_ref, blv_ref,
               wd1_ref, bd1_ref, wd2_ref, bd2_ref, eps_ref,
               z_ref, kl_ref):
    h = h_ref[...]
    mm = functools.partial(jnp.dot, preferred_element_type=jnp.float32,
                           precision=lax.Precision.HIGHEST)
    mu = mm(h, wmu_ref[...]) + bmu_ref[...]
    lv = mm(h, wlv_ref[...]) + blv_ref[...]
    elv = jnp.exp(lv)
    kl_terms = 1.0 + 2.0 * lv - mu * mu - elv * elv
    kl = -0.5 * (jnp.sum(kl_terms) / jnp.float32(N))
    kl_ref[...] = jnp.full((8, 128), kl, dtype=jnp.float32)
    z_lat = eps_ref[...] * elv + mu
    z1 = jnp.maximum(mm(z_lat, wd1_ref[...]) + bd1_ref[...], 0.0)
    z_ref[...] = mm(z1, wd2_ref[...]) + bd2_ref[...]


_head_tc = pl.pallas_call(
    _head_body,
    out_shape=(
        jax.ShapeDtypeStruct((N, D), jnp.float32),
        jax.ShapeDtypeStruct((8, 128), jnp.float32),
    ),
)


def _loss_body(pv_ref, nv_ref, o_ref):
    n2 = jnp.float32(N) * jnp.float32(N)
    pos_weight = (n2 - 2.0) / 2.0
    norm = n2 / ((n2 - 2.0) * 2.0)
    pos_adj = jax.nn.sigmoid(pv_ref[...])
    pos_loss = -jnp.mean(jnp.log(pos_adj + EPS))
    neg_adj = jax.nn.sigmoid(nv_ref[...])
    neg_loss = -jnp.mean(jnp.log(1.0 - neg_adj + EPS))
    recon = norm * (pos_loss * pos_weight + neg_loss)
    o_ref[...] = jnp.full((8, 128), recon, dtype=jnp.float32)


_loss_tc = pl.pallas_call(
    _loss_body,
    out_shape=jax.ShapeDtypeStruct((8, 128), jnp.float32),
)


def kernel(x, edge_index, W0, b0, W1, b1, W2, b2, Wmu, bmu, Wlv, blv,
           Wd1, bd1, Wd2, bd2, eps, neg_edge_index):
    src_rs = edge_index[0].reshape(NW, NCH, C)
    dst_rs = edge_index[1].reshape(NW, NCH, C)
    zeros = jnp.zeros((N, D), jnp.float32)

    h = x
    for W, b in ((W0, b0), (W1, b1), (W2, b2)):
        parts = _segsum(h, src_rs, dst_rs, zeros)
        h = _layer_tc(h, parts, W, b.reshape(1, D))

    z, klbuf = _head_tc(h, Wmu, bmu.reshape(1, D), Wlv, blv.reshape(1, D),
                        Wd1, bd1.reshape(1, D), Wd2, bd2.reshape(1, D), eps)

    eidx = jnp.stack([edge_index, neg_edge_index]).reshape(2, 2, NW, NCH, C)
    vals = _edge_dots(z, eidx)
    lossbuf = _loss_tc(vals[0].reshape(2500, 128), vals[1].reshape(2500, 128))
    return (lossbuf[0, 0], jnp.float32(0.0), klbuf[0, 0])


# speedup vs baseline: 4.1183x; 4.1183x over previous
import functools

import jax
import jax.numpy as jnp
from jax import lax
from jax.experimental import pallas as pl
from jax.experimental.pallas import tpu as pltpu
from jax.experimental.pallas import tpu_sc as plsc

N = 10000
E = 320000
D = 128
EPS = 1e-15

NC = 2
NS = 16
NW = NC * NS
EW = E // NW
C = 80
NCH = EW // C
RPT = 632
NP = NS * RPT

_mesh = plsc.VectorSubcoreMesh(core_axis_name="c", subcore_axis_name="s")


def _segsum_body(h_hbm, src_hbm, dst_hbm, zeros_hbm, out_hbm,
                 sidx_v, didx_v, rows_v, acc_sh, sem):
    c = lax.axis_index("c")
    s = lax.axis_index("s")
    wid = s * NC + c
    base = s * RPT
    pltpu.sync_copy(zeros_hbm.at[pl.ds(base, RPT)], acc_sh.at[pl.ds(base, RPT)])
    plsc.subcore_barrier()
    pltpu.sync_copy(src_hbm.at[wid], sidx_v)
    pltpu.sync_copy(dst_hbm.at[wid], didx_v)

    def chunk(j, carry):
        pltpu.async_copy(h_hbm.at[sidx_v.at[j]], rows_v, sem).wait()
        pltpu.sync_copy(rows_v, acc_sh.at[didx_v.at[j]], add=True)
        return carry

    lax.fori_loop(0, NCH, chunk, 0)
    plsc.subcore_barrier()
    pltpu.sync_copy(acc_sh.at[pl.ds(base, RPT)], out_hbm.at[c, pl.ds(base, RPT)])


_segsum = pl.kernel(
    _segsum_body,
    out_type=jax.ShapeDtypeStruct((NC, NP, D), jnp.float32),
    mesh=_mesh,
    scratch_types=[
        pltpu.VMEM((NCH, C), jnp.int32),
        pltpu.VMEM((NCH, C), jnp.int32),
        pltpu.VMEM((C, D), jnp.float32),
        pltpu.VMEM_SHARED((NP, D), jnp.float32),
        pltpu.SemaphoreType.DMA,
    ],
)


def _edge_dots_body(z_hbm, eidx_hbm, out_hbm,
                    sidx_v, didx_v, zs_v, zd_v, vals_v, fold_v, place_v,
                    sem_s, sem_d):
    c = lax.axis_index("c")
    s = lax.axis_index("s")
    wid = s * NC + c

    for p in range(2):
        pltpu.sync_copy(eidx_hbm.at[p, 0, wid], sidx_v)
        pltpu.sync_copy(eidx_hbm.at[p, 1, wid], didx_v)

        def chunk(j, carry):
            cs = pltpu.async_copy(z_hbm.at[sidx_v.at[j]], zs_v, sem_s)
            cd = pltpu.async_copy(z_hbm.at[didx_v.at[j]], zd_v, sem_d)
            cs.wait()
            cd.wait()

            def group(g, carry2):
                for e in range(16):
                    row = g * 16 + e
                    acc = zs_v[row, pl.ds(0, 16)] * zd_v[row, pl.ds(0, 16)]
                    for q in range(1, D // 16):
                        acc = acc + (zs_v[row, pl.ds(q * 16, 16)] *
                                     zd_v[row, pl.ds(q * 16, 16)])
                    v = acc
                    for off in (8, 4, 2, 1):
                        fold_v[pl.ds(0, 16)] = v
                        fold_v[pl.ds(16, 16)] = v
                        v = v + fold_v[pl.ds(off, 16)]
                    place_v[pl.ds(e, 16)] = v
                vals_v[j, pl.ds(pl.multiple_of(g * 16, 16), 16)] = (
                    place_v[pl.ds(0, 16)])
                return carry2

            lax.fori_loop(0, C // 16, group, 0)
            return carry

        lax.fori_loop(0, NCH, chunk, 0)
        pltpu.sync_copy(vals_v, out_hbm.at[p, wid])


_edge_dots = pl.kernel(
    _edge_dots_body,
    out_type=jax.ShapeDtypeStruct((2, NW, NCH, C), jnp.float32),
    mesh=_mesh,
    scratch_types=[
        pltpu.VMEM((NCH, C), jnp.int32),
        pltpu.VMEM((NCH, C), jnp.int32),
        pltpu.VMEM((C, D), jnp.float32),
        pltpu.VMEM((C, D), jnp.float32),
        pltpu.VMEM((NCH, C), jnp.float32),
        pltpu.VMEM((32,), jnp.float32),
        pltpu.VMEM((32,), jnp.float32),
        pltpu.SemaphoreType.DMA,
        pltpu.SemaphoreType.DMA,
    ],
)


def _layer_body(h_ref, m_ref, w_ref, b_ref, o_ref):
    hm = h_ref[...] + m_ref[0, :N] + m_ref[1, :N]
    y = jnp.dot(hm, w_ref[...], preferred_element_type=jnp.float32,
                precision=lax.Precision.HIGHEST)
    o_ref[...] = jnp.maximum(y + b_ref[...], 0.0)


_layer_tc = pl.pallas_call(
    _layer_body,
    out_shape=jax.ShapeDtypeStruct((N, D), jnp.float32),
)


def _head_body(h_ref, wmu_ref, bmu_ref, wlv_ref, blv_ref,
               wd1_ref, bd1_ref, wd2_ref, bd2_ref, eps_ref,
               z_ref, kl_ref):
    h = h_ref[...]
    mm = functools.partial(jnp.dot, preferred_element_type=jnp.float32,
                           precision=lax.Precision.HIGHEST)
    mu = mm(h, wmu_ref[...]) + bmu_ref[...]
    lv = mm(h, wlv_ref[...]) + blv_ref[...]
    elv = jnp.exp(lv)
    kl_terms = 1.0 + 2.0 * lv - mu * mu - elv * elv
    kl = -0.5 * (jnp.sum(kl_terms) / jnp.float32(N))
    kl_ref[...] = jnp.full((8, 128), kl, dtype=jnp.float32)
    z_lat = eps_ref[...] * elv + mu
    z1 = jnp.maximum(mm(z_lat, wd1_ref[...]) + bd1_ref[...], 0.0)
    z_ref[...] = mm(z1, wd2_ref[...]) + bd2_ref[...]


_head_tc = pl.pallas_call(
    _head_body,
    out_shape=(
        jax.ShapeDtypeStruct((N, D), jnp.float32),
        jax.ShapeDtypeStruct((8, 128), jnp.float32),
    ),
)


def _loss_body(pv_ref, nv_ref, o_ref):
    n2 = jnp.float32(N) * jnp.float32(N)
    pos_weight = (n2 - 2.0) / 2.0
    norm = n2 / ((n2 - 2.0) * 2.0)
    pos_adj = jax.nn.sigmoid(pv_ref[...])
    pos_loss = -jnp.mean(jnp.log(pos_adj + EPS))
    neg_adj = jax.nn.sigmoid(nv_ref[...])
    neg_loss = -jnp.mean(jnp.log(1.0 - neg_adj + EPS))
    recon = norm * (pos_loss * pos_weight + neg_loss)
    o_ref[...] = jnp.full((8, 128), recon, dtype=jnp.float32)


_loss_tc = pl.pallas_call(
    _loss_body,
    out_shape=jax.ShapeDtypeStruct((8, 128), jnp.float32),
)


def kernel(x, edge_index, W0, b0, W1, b1, W2, b2, Wmu, bmu, Wlv, blv,
           Wd1, bd1, Wd2, bd2, eps, neg_edge_index):
    src_rs = edge_index[0].reshape(NW, NCH, C)
    dst_rs = edge_index[1].reshape(NW, NCH, C)
    zeros = jnp.zeros((NP, D), jnp.float32)

    h = x
    for W, b in ((W0, b0), (W1, b1), (W2, b2)):
        parts = _segsum(h, src_rs, dst_rs, zeros)
        h = _layer_tc(h, parts, W, b.reshape(1, D))

    z, klbuf = _head_tc(h, Wmu, bmu.reshape(1, D), Wlv, blv.reshape(1, D),
                        Wd1, bd1.reshape(1, D), Wd2, bd2.reshape(1, D), eps)

    eidx = jnp.stack([edge_index, neg_edge_index]).reshape(2, 2, NW, NCH, C)
    vals = _edge_dots(z, eidx)
    lossbuf = _loss_tc(vals[0].reshape(2500, 128), vals[1].reshape(2500, 128))
    return (lossbuf[0, 0], jnp.float32(0.0), klbuf[0, 0])


# speedup vs baseline: 5.0577x; 1.2281x over previous
import functools

import jax
import jax.numpy as jnp
from jax import lax
from jax.experimental import pallas as pl
from jax.experimental.pallas import tpu as pltpu
from jax.experimental.pallas import tpu_sc as plsc

N = 10000
E = 320000
D = 128
EPS = 1e-15

NC = 2
NS = 16
NW = NC * NS
EW = E // NW
C = 80
NCH = EW // C
CS = 80
NCHS = EW // CS
RPT = 632
NP = NS * RPT

_mesh = plsc.VectorSubcoreMesh(core_axis_name="c", subcore_axis_name="s")


def _segsum_body(h_hbm, src_hbm, dst_hbm, zeros_hbm, out_hbm,
                 sidx_v, didx_v, rows_a, acc_sh, sem_a):
    c = lax.axis_index("c")
    s = lax.axis_index("s")
    wid = s * NC + c
    base = s * RPT
    pltpu.sync_copy(zeros_hbm, acc_sh.at[pl.ds(base, RPT)])
    plsc.subcore_barrier()
    pltpu.sync_copy(src_hbm.at[wid], sidx_v)
    pltpu.sync_copy(dst_hbm.at[wid], didx_v)

    def chunk(j, carry):
        pltpu.async_copy(h_hbm.at[sidx_v.at[j]], rows_a, sem_a).wait()
        pltpu.sync_copy(rows_a, acc_sh.at[didx_v.at[j]], add=True)
        return carry

    lax.fori_loop(0, NCHS, chunk, 0)
    plsc.subcore_barrier()
    pltpu.sync_copy(acc_sh.at[pl.ds(base, RPT)], out_hbm.at[c, pl.ds(base, RPT)])


_segsum = pl.kernel(
    _segsum_body,
    out_type=jax.ShapeDtypeStruct((NC, NP, D), jnp.float32),
    mesh=_mesh,
    scratch_types=[
        pltpu.VMEM((NCHS, CS), jnp.int32),
        pltpu.VMEM((NCHS, CS), jnp.int32),
        pltpu.VMEM((CS, D), jnp.float32),
        pltpu.VMEM_SHARED((NP, D), jnp.float32),
        pltpu.SemaphoreType.DMA,
    ],
)


def _edge_dots_body(z_hbm, eidx_hbm, out_hbm,
                    sidx_v, didx_v, zs_a, zd_a, zs_b, zd_b, vals_v,
                    fold_v, place_v,
                    sem_sa, sem_da, sem_sb, sem_db):
    c = lax.axis_index("c")
    s = lax.axis_index("s")
    wid = s * NC + c

    def fire(j, zs, zd, ss, sd):
        pltpu.async_copy(z_hbm.at[sidx_v.at[j]], zs, ss)
        pltpu.async_copy(z_hbm.at[didx_v.at[j]], zd, sd)

    def drain(j, zs, zd, ss, sd):
        pltpu.make_async_copy(z_hbm.at[sidx_v.at[j]], zs, ss).wait()
        pltpu.make_async_copy(z_hbm.at[didx_v.at[j]], zd, sd).wait()

    def compute(j, zs, zd):
        def group(g, carry2):
            for e in range(16):
                row = g * 16 + e
                acc = zs[row, pl.ds(0, 16)] * zd[row, pl.ds(0, 16)]
                for q in range(1, D // 16):
                    acc = acc + (zs[row, pl.ds(q * 16, 16)] *
                                 zd[row, pl.ds(q * 16, 16)])
                v = acc
                for off in (8, 4, 2, 1):
                    fold_v[pl.ds(0, 16)] = v
                    fold_v[pl.ds(16, 16)] = v
                    v = v + fold_v[pl.ds(off, 16)]
                place_v[pl.ds(e, 16)] = v
            vals_v[j, pl.ds(pl.multiple_of(g * 16, 16), 16)] = (
                place_v[pl.ds(0, 16)])
            return carry2

        lax.fori_loop(0, C // 16, group, 0)

    for p in range(2):
        pltpu.sync_copy(eidx_hbm.at[p, 0, wid], sidx_v)
        pltpu.sync_copy(eidx_hbm.at[p, 1, wid], didx_v)

        fire(0, zs_a, zd_a, sem_sa, sem_da)

        def pair(jj, carry):
            j = 2 * jj
            drain(j, zs_a, zd_a, sem_sa, sem_da)
            fire(j + 1, zs_b, zd_b, sem_sb, sem_db)
            compute(j, zs_a, zd_a)
            drain(j + 1, zs_b, zd_b, sem_sb, sem_db)
            fire(j + 2, zs_a, zd_a, sem_sa, sem_da)
            compute(j + 1, zs_b, zd_b)
            return carry

        lax.fori_loop(0, NCH // 2, pair, 0)
        drain(NCH - 1, zs_a, zd_a, sem_sa, sem_da)
        compute(NCH - 1, zs_a, zd_a)
        pltpu.sync_copy(vals_v, out_hbm.at[p, wid])


_edge_dots = pl.kernel(
    _edge_dots_body,
    out_type=jax.ShapeDtypeStruct((2, NW, NCH, C), jnp.float32),
    mesh=_mesh,
    scratch_types=[
        pltpu.VMEM((NCH, C), jnp.int32),
        pltpu.VMEM((NCH, C), jnp.int32),
        pltpu.VMEM((C, D), jnp.float32),
        pltpu.VMEM((C, D), jnp.float32),
        pltpu.VMEM((C, D), jnp.float32),
        pltpu.VMEM((C, D), jnp.float32),
        pltpu.VMEM((NCH, C), jnp.float32),
        pltpu.VMEM((32,), jnp.float32),
        pltpu.VMEM((32,), jnp.float32),
        pltpu.SemaphoreType.DMA,
        pltpu.SemaphoreType.DMA,
        pltpu.SemaphoreType.DMA,
        pltpu.SemaphoreType.DMA,
    ],
)


def _layer_body(h_ref, m_ref, w_ref, b_ref, o_ref):
    hm = h_ref[...] + m_ref[0, :N] + m_ref[1, :N]
    y = jnp.dot(hm, w_ref[...], preferred_element_type=jnp.float32,
                precision=lax.Precision.HIGHEST)
    o_ref[...] = jnp.maximum(y + b_ref[...], 0.0)


_layer_tc = pl.pallas_call(
    _layer_body,
    out_shape=jax.ShapeDtypeStruct((N, D), jnp.float32),
)


def _head_body(h_ref, wmu_ref, bmu_ref, wlv_ref, blv_ref,
               wd1_ref, bd1_ref, wd2_ref, bd2_ref, eps_ref,
               z_ref, kl_ref):
    h = h_ref[...]
    mm = functools.partial(jnp.dot, preferred_element_type=jnp.float32,
                           precision=lax.Precision.HIGHEST)
    mu = mm(h, wmu_ref[...]) + bmu_ref[...]
    lv = mm(h, wlv_ref[...]) + blv_ref[...]
    elv = jnp.exp(lv)
    kl_terms = 1.0 + 2.0 * lv - mu * mu - elv * elv
    kl = -0.5 * (jnp.sum(kl_terms) / jnp.float32(N))
    kl_ref[...] = jnp.full((8, 128), kl, dtype=jnp.float32)
    z_lat = eps_ref[...] * elv + mu
    z1 = jnp.maximum(mm(z_lat, wd1_ref[...]) + bd1_ref[...], 0.0)
    z_ref[...] = mm(z1, wd2_ref[...]) + bd2_ref[...]


_head_tc = pl.pallas_call(
    _head_body,
    out_shape=(
        jax.ShapeDtypeStruct((N, D), jnp.float32),
        jax.ShapeDtypeStruct((8, 128), jnp.float32),
    ),
)


def _loss_body(pv_ref, nv_ref, o_ref):
    n2 = jnp.float32(N) * jnp.float32(N)
    pos_weight = (n2 - 2.0) / 2.0
    norm = n2 / ((n2 - 2.0) * 2.0)
    pos_adj = jax.nn.sigmoid(pv_ref[...])
    pos_loss = -jnp.mean(jnp.log(pos_adj + EPS))
    neg_adj = jax.nn.sigmoid(nv_ref[...])
    neg_loss = -jnp.mean(jnp.log(1.0 - neg_adj + EPS))
    recon = norm * (pos_loss * pos_weight + neg_loss)
    o_ref[...] = jnp.full((8, 128), recon, dtype=jnp.float32)


_loss_tc = pl.pallas_call(
    _loss_body,
    out_shape=jax.ShapeDtypeStruct((8, 128), jnp.float32),
)


def kernel(x, edge_index, W0, b0, W1, b1, W2, b2, Wmu, bmu, Wlv, blv,
           Wd1, bd1, Wd2, bd2, eps, neg_edge_index):
    src_rs = edge_index[0].reshape(NW, NCHS, CS)
    dst_rs = edge_index[1].reshape(NW, NCHS, CS)
    zeros = jnp.zeros((RPT, D), jnp.float32)

    h = x
    for W, b in ((W0, b0), (W1, b1), (W2, b2)):
        parts = _segsum(h, src_rs, dst_rs, zeros)
        h = _layer_tc(h, parts, W, b.reshape(1, D))

    z, klbuf = _head_tc(h, Wmu, bmu.reshape(1, D), Wlv, blv.reshape(1, D),
                        Wd1, bd1.reshape(1, D), Wd2, bd2.reshape(1, D), eps)

    eidx = jnp.stack([edge_index, neg_edge_index]).reshape(2, 2, NW, NCH, C)
    vals = _edge_dots(z, eidx)
    lossbuf = _loss_tc(vals[0].reshape(2500, 128), vals[1].reshape(2500, 128))
    return (lossbuf[0, 0], jnp.float32(0.0), klbuf[0, 0])


# speedup vs baseline: 6.1156x; 1.2092x over previous
import functools

import jax
import jax.numpy as jnp
from jax import lax
from jax.experimental import pallas as pl
from jax.experimental.pallas import tpu as pltpu
from jax.experimental.pallas import tpu_sc as plsc

N = 10000
E = 320000
D = 128
EPS = 1e-15

NC = 2
NS = 16
NW = NC * NS
EW = E // NW
C = 80
NCH = EW // C
CS = 80
NCHS = EW // CS
RPT = 632
NP = NS * RPT

_mesh = plsc.VectorSubcoreMesh(core_axis_name="c", subcore_axis_name="s")


def _segsum_body(h_hbm, src_hbm, dst_hbm, zeros_hbm, out_hbm,
                 sidx_v, didx_v, rows0, acc_sh, sem0):
    c = lax.axis_index("c")
    s = lax.axis_index("s")
    wid = s * NC + c
    base = s * RPT
    pltpu.sync_copy(zeros_hbm, acc_sh.at[pl.ds(base, RPT)])
    plsc.subcore_barrier()
    pltpu.sync_copy(src_hbm.at[wid], sidx_v)
    pltpu.sync_copy(dst_hbm.at[wid], didx_v)

    def chunk(j, carry):
        pltpu.async_copy(h_hbm.at[sidx_v.at[j]], rows0, sem0).wait()
        pltpu.sync_copy(rows0, acc_sh.at[didx_v.at[j]], add=True)
        return carry

    lax.fori_loop(0, NCHS, chunk, 0)
    plsc.subcore_barrier()
    pltpu.sync_copy(acc_sh.at[pl.ds(base, RPT)], out_hbm.at[c, pl.ds(base, RPT)])


_segsum = pl.kernel(
    _segsum_body,
    out_type=jax.ShapeDtypeStruct((NC, NP, D), jnp.float32),
    mesh=_mesh,
    scratch_types=[
        pltpu.VMEM((NCHS, CS), jnp.int32),
        pltpu.VMEM((NCHS, CS), jnp.int32),
        pltpu.VMEM((CS, D), jnp.float32),
        pltpu.VMEM_SHARED((NP, D), jnp.float32),
        pltpu.SemaphoreType.DMA,
    ],
)


def _edge_dots_body(z_hbm, eidx_hbm, out_hbm,
                    sidx_v, didx_v, zs_a, zd_a, zs_b, zd_b, vals_v,
                    fold_v, place_v, fold_w, place_w,
                    sem_sa, sem_da, sem_sb, sem_db):
    c = lax.axis_index("c")
    s = lax.axis_index("s")
    wid = s * NC + c
    low8 = lax.iota(jnp.int32, 16) < 8

    def fire(j, zs, zd, ss, sd):
        pltpu.async_copy(z_hbm.at[sidx_v.at[j]], zs, ss)
        pltpu.async_copy(z_hbm.at[didx_v.at[j]], zd, sd)

    def drain(j, zs, zd, ss, sd):
        pltpu.make_async_copy(z_hbm.at[sidx_v.at[j]], zs, ss).wait()
        pltpu.make_async_copy(z_hbm.at[didx_v.at[j]], zd, sd).wait()

    def mac(zs, zd, row):
        acc = zs[row, pl.ds(0, 16)] * zd[row, pl.ds(0, 16)]
        for q in range(1, D // 16):
            acc = acc + (zs[row, pl.ds(q * 16, 16)] *
                         zd[row, pl.ds(q * 16, 16)])
        return acc

    def compute(j, zs, zd):
        def group(g, carry2):
            for e in range(8):
                va = mac(zs, zd, g * 16 + e)
                vb = mac(zs, zd, g * 16 + e + 8)
                for off in (8, 4, 2, 1):
                    fold_v[pl.ds(0, 16)] = va
                    fold_v[pl.ds(16, 16)] = va
                    va = va + fold_v[pl.ds(off, 16)]
                    fold_w[pl.ds(0, 16)] = vb
                    fold_w[pl.ds(16, 16)] = vb
                    vb = vb + fold_w[pl.ds(off, 16)]
                place_v[pl.ds(e, 16)] = va
                place_w[pl.ds(e, 16)] = vb
            pv = place_v[pl.ds(0, 16)]
            pw = place_w[pl.ds(0, 16)]
            fold_v[pl.ds(0, 16)] = pw
            fold_v[pl.ds(16, 16)] = pw
            pwr = fold_v[pl.ds(8, 16)]
            vals_v[j, pl.ds(pl.multiple_of(g * 16, 16), 16)] = (
                jnp.where(low8, pv, pwr))
            return carry2

        lax.fori_loop(0, C // 16, group, 0)

    for p in range(2):
        pltpu.sync_copy(eidx_hbm.at[p, 0, wid], sidx_v)
        pltpu.sync_copy(eidx_hbm.at[p, 1, wid], didx_v)

        fire(0, zs_a, zd_a, sem_sa, sem_da)

        def pair(jj, carry):
            j = 2 * jj
            drain(j, zs_a, zd_a, sem_sa, sem_da)
            fire(j + 1, zs_b, zd_b, sem_sb, sem_db)
            compute(j, zs_a, zd_a)
            drain(j + 1, zs_b, zd_b, sem_sb, sem_db)
            fire(j + 2, zs_a, zd_a, sem_sa, sem_da)
            compute(j + 1, zs_b, zd_b)
            return carry

        lax.fori_loop(0, NCH // 2, pair, 0)
        drain(NCH - 1, zs_a, zd_a, sem_sa, sem_da)
        compute(NCH - 1, zs_a, zd_a)
        pltpu.sync_copy(vals_v, out_hbm.at[p, wid])


_edge_dots = pl.kernel(
    _edge_dots_body,
    out_type=jax.ShapeDtypeStruct((2, NW, NCH, C), jnp.float32),
    mesh=_mesh,
    scratch_types=[
        pltpu.VMEM((NCH, C), jnp.int32),
        pltpu.VMEM((NCH, C), jnp.int32),
        pltpu.VMEM((C, D), jnp.float32),
        pltpu.VMEM((C, D), jnp.float32),
        pltpu.VMEM((C, D), jnp.float32),
        pltpu.VMEM((C, D), jnp.float32),
        pltpu.VMEM((NCH, C), jnp.float32),
        pltpu.VMEM((32,), jnp.float32),
        pltpu.VMEM((32,), jnp.float32),
        pltpu.VMEM((32,), jnp.float32),
        pltpu.VMEM((32,), jnp.float32),
        pltpu.SemaphoreType.DMA,
        pltpu.SemaphoreType.DMA,
        pltpu.SemaphoreType.DMA,
        pltpu.SemaphoreType.DMA,
    ],
)


def _layer_body(h_ref, m_ref, w_ref, b_ref, o_ref):
    hm = h_ref[...] + m_ref[0, :N] + m_ref[1, :N]
    y = jnp.dot(hm, w_ref[...], preferred_element_type=jnp.float32,
                precision=lax.Precision.HIGHEST)
    o_ref[...] = jnp.maximum(y + b_ref[...], 0.0)


_layer_tc = pl.pallas_call(
    _layer_body,
    out_shape=jax.ShapeDtypeStruct((N, D), jnp.float32),
)


def _head_body(h_ref, wmu_ref, bmu_ref, wlv_ref, blv_ref,
               wd1_ref, bd1_ref, wd2_ref, bd2_ref, eps_ref,
               z_ref, kl_ref):
    h = h_ref[...]
    mm = functools.partial(jnp.dot, preferred_element_type=jnp.float32,
                           precision=lax.Precision.HIGHEST)
    mu = mm(h, wmu_ref[...]) + bmu_ref[...]
    lv = mm(h, wlv_ref[...]) + blv_ref[...]
    elv = jnp.exp(lv)
    kl_terms = 1.0 + 2.0 * lv - mu * mu - elv * elv
    kl = -0.5 * (jnp.sum(kl_terms) / jnp.float32(N))
    kl_ref[...] = jnp.full((8, 128), kl, dtype=jnp.float32)
    z_lat = eps_ref[...] * elv + mu
    z1 = jnp.maximum(mm(z_lat, wd1_ref[...]) + bd1_ref[...], 0.0)
    z_ref[...] = mm(z1, wd2_ref[...]) + bd2_ref[...]


_head_tc = pl.pallas_call(
    _head_body,
    out_shape=(
        jax.ShapeDtypeStruct((N, D), jnp.float32),
        jax.ShapeDtypeStruct((8, 128), jnp.float32),
    ),
)


def _loss_body(pv_ref, nv_ref, o_ref):
    n2 = jnp.float32(N) * jnp.float32(N)
    pos_weight = (n2 - 2.0) / 2.0
    norm = n2 / ((n2 - 2.0) * 2.0)
    pos_adj = jax.nn.sigmoid(pv_ref[...])
    pos_loss = -jnp.mean(jnp.log(pos_adj + EPS))
    neg_adj = jax.nn.sigmoid(nv_ref[...])
    neg_loss = -jnp.mean(jnp.log(1.0 - neg_adj + EPS))
    recon = norm * (pos_loss * pos_weight + neg_loss)
    o_ref[...] = jnp.full((8, 128), recon, dtype=jnp.float32)


_loss_tc = pl.pallas_call(
    _loss_body,
    out_shape=jax.ShapeDtypeStruct((8, 128), jnp.float32),
)


def kernel(x, edge_index, W0, b0, W1, b1, W2, b2, Wmu, bmu, Wlv, blv,
           Wd1, bd1, Wd2, bd2, eps, neg_edge_index):
    src_rs = edge_index[0].reshape(NW, NCHS, CS)
    dst_rs = edge_index[1].reshape(NW, NCHS, CS)
    zeros = jnp.zeros((RPT, D), jnp.float32)

    h = x
    for W, b in ((W0, b0), (W1, b1), (W2, b2)):
        parts = _segsum(h, src_rs, dst_rs, zeros)
        h = _layer_tc(h, parts, W, b.reshape(1, D))

    z, klbuf = _head_tc(h, Wmu, bmu.reshape(1, D), Wlv, blv.reshape(1, D),
                        Wd1, bd1.reshape(1, D), Wd2, bd2.reshape(1, D), eps)

    eidx = jnp.stack([edge_index, neg_edge_index]).reshape(2, 2, NW, NCH, C)
    vals = _edge_dots(z, eidx)
    lossbuf = _loss_tc(vals[0].reshape(2500, 128), vals[1].reshape(2500, 128))
    return (lossbuf[0, 0], jnp.float32(0.0), klbuf[0, 0])


# speedup vs baseline: 7.0700x; 1.1560x over previous
import functools

import jax
import jax.numpy as jnp
from jax import lax
from jax.experimental import pallas as pl
from jax.experimental.pallas import tpu as pltpu
from jax.experimental.pallas import tpu_sc as plsc

N = 10000
E = 320000
D = 128
EPS = 1e-15

NC = 2
NS = 16
NW = NC * NS
EW = E // NW
C = 80
NCH = EW // C
CS = 80
NCHS = EW // CS
RPT = 632
NP = NS * RPT

_mesh = plsc.VectorSubcoreMesh(core_axis_name="c", subcore_axis_name="s")


def _segsum_body(h_hbm, packed_hbm, zeros_hbm, out_hbm,
                 pidx_v, sa_v, da_v, sb_v, db_v, rows_a, rows_b, acc_sh,
                 sem_a, sem_b):
    c = lax.axis_index("c")
    s = lax.axis_index("s")
    wid = s * NC + c
    base = s * RPT
    pltpu.sync_copy(zeros_hbm, acc_sh.at[pl.ds(base, RPT)])
    plsc.subcore_barrier()
    pltpu.sync_copy(packed_hbm.at[wid], pidx_v)

    def unpack(j, sbuf, dbuf):
        for k in range(CS // 16):
            pk = pidx_v[j, pl.ds(k * 16, 16)]
            sbuf[pl.ds(pl.multiple_of(k * 16, 16), 16)] = pk & 0xFFFF
            dbuf[pl.ds(pl.multiple_of(k * 16, 16), 16)] = (
                lax.shift_right_logical(pk, 16))

    unpack(0, sa_v, da_v)
    pltpu.async_copy(h_hbm.at[sa_v], rows_a, sem_a)

    def pair(jj, carry):
        j = 2 * jj
        unpack(j + 1, sb_v, db_v)
        pltpu.make_async_copy(h_hbm.at[sa_v], rows_a, sem_a).wait()
        pltpu.async_copy(h_hbm.at[sb_v], rows_b, sem_b)
        pltpu.sync_copy(rows_a, acc_sh.at[da_v], add=True)
        unpack(j + 2, sa_v, da_v)
        pltpu.make_async_copy(h_hbm.at[sb_v], rows_b, sem_b).wait()
        pltpu.async_copy(h_hbm.at[sa_v], rows_a, sem_a)
        pltpu.sync_copy(rows_b, acc_sh.at[db_v], add=True)
        return carry

    lax.fori_loop(0, NCHS // 2, pair, 0)
    pltpu.make_async_copy(h_hbm.at[sa_v], rows_a, sem_a).wait()
    pltpu.sync_copy(rows_a, acc_sh.at[da_v], add=True)
    plsc.subcore_barrier()
    pltpu.sync_copy(acc_sh.at[pl.ds(base, RPT)], out_hbm.at[c, pl.ds(base, RPT)])


_segsum = pl.kernel(
    _segsum_body,
    out_type=jax.ShapeDtypeStruct((NC, NP, D), jnp.float32),
    mesh=_mesh,
    scratch_types=[
        pltpu.VMEM((NCHS, CS), jnp.int32),
        pltpu.VMEM((CS,), jnp.int32),
        pltpu.VMEM((CS,), jnp.int32),
        pltpu.VMEM((CS,), jnp.int32),
        pltpu.VMEM((CS,), jnp.int32),
        pltpu.VMEM((CS, D), jnp.float32),
        pltpu.VMEM((CS, D), jnp.float32),
        pltpu.VMEM_SHARED((NP, D), jnp.float32),
        pltpu.SemaphoreType.DMA,
        pltpu.SemaphoreType.DMA,
    ],
)


def _edge_dots_body(z_hbm, eidx_hbm, out_hbm,
                    sidx_v, didx_v, zs_a, zd_a, zs_b, zd_b, vals_v,
                    fold_v, place_v, fold_w, place_w,
                    sem_sa, sem_da, sem_sb, sem_db):
    c = lax.axis_index("c")
    s = lax.axis_index("s")
    wid = s * NC + c
    low8 = lax.iota(jnp.int32, 16) < 8

    def fire(j, zs, zd, ss, sd):
        pltpu.async_copy(z_hbm.at[sidx_v.at[j]], zs, ss)
        pltpu.async_copy(z_hbm.at[didx_v.at[j]], zd, sd)

    def drain(j, zs, zd, ss, sd):
        pltpu.make_async_copy(z_hbm.at[sidx_v.at[j]], zs, ss).wait()
        pltpu.make_async_copy(z_hbm.at[didx_v.at[j]], zd, sd).wait()

    def mac(zs, zd, row):
        acc = zs[row, pl.ds(0, 16)] * zd[row, pl.ds(0, 16)]
        for q in range(1, D // 16):
            acc = acc + (zs[row, pl.ds(q * 16, 16)] *
                         zd[row, pl.ds(q * 16, 16)])
        return acc

    def compute(j, zs, zd):
        def group(g, carry2):
            for e in range(8):
                va = mac(zs, zd, g * 16 + e)
                vb = mac(zs, zd, g * 16 + e + 8)
                for off in (8, 4, 2, 1):
                    fold_v[pl.ds(0, 16)] = va
                    fold_v[pl.ds(16, 16)] = va
                    va = va + fold_v[pl.ds(off, 16)]
                    fold_w[pl.ds(0, 16)] = vb
                    fold_w[pl.ds(16, 16)] = vb
                    vb = vb + fold_w[pl.ds(off, 16)]
                place_v[pl.ds(e, 16)] = va
                place_w[pl.ds(e, 16)] = vb
            pv = place_v[pl.ds(0, 16)]
            pw = place_w[pl.ds(0, 16)]
            fold_v[pl.ds(0, 16)] = pw
            fold_v[pl.ds(16, 16)] = pw
            pwr = fold_v[pl.ds(8, 16)]
            vals_v[j, pl.ds(pl.multiple_of(g * 16, 16), 16)] = (
                jnp.where(low8, pv, pwr))
            return carry2

        lax.fori_loop(0, C // 16, group, 0)

    for p in range(2):
        pltpu.sync_copy(eidx_hbm.at[p, 0, wid], sidx_v)
        pltpu.sync_copy(eidx_hbm.at[p, 1, wid], didx_v)

        fire(0, zs_a, zd_a, sem_sa, sem_da)

        def pair(jj, carry):
            j = 2 * jj
            drain(j, zs_a, zd_a, sem_sa, sem_da)
            fire(j + 1, zs_b, zd_b, sem_sb, sem_db)
            compute(j, zs_a, zd_a)
            drain(j + 1, zs_b, zd_b, sem_sb, sem_db)
            fire(j + 2, zs_a, zd_a, sem_sa, sem_da)
            compute(j + 1, zs_b, zd_b)
            return carry

        lax.fori_loop(0, NCH // 2, pair, 0)
        drain(NCH - 1, zs_a, zd_a, sem_sa, sem_da)
        compute(NCH - 1, zs_a, zd_a)
        pltpu.sync_copy(vals_v, out_hbm.at[p, wid])


_edge_dots = pl.kernel(
    _edge_dots_body,
    out_type=jax.ShapeDtypeStruct((2, NW, NCH, C), jnp.float32),
    mesh=_mesh,
    scratch_types=[
        pltpu.VMEM((NCH, C), jnp.int32),
        pltpu.VMEM((NCH, C), jnp.int32),
        pltpu.VMEM((C, D), jnp.float32),
        pltpu.VMEM((C, D), jnp.float32),
        pltpu.VMEM((C, D), jnp.float32),
        pltpu.VMEM((C, D), jnp.float32),
        pltpu.VMEM((NCH, C), jnp.float32),
        pltpu.VMEM((32,), jnp.float32),
        pltpu.VMEM((32,), jnp.float32),
        pltpu.VMEM((32,), jnp.float32),
        pltpu.VMEM((32,), jnp.float32),
        pltpu.SemaphoreType.DMA,
        pltpu.SemaphoreType.DMA,
        pltpu.SemaphoreType.DMA,
        pltpu.SemaphoreType.DMA,
    ],
)


def _layer_body(h_ref, m_ref, w_ref, b_ref, o_ref):
    hm = h_ref[...] + m_ref[0, :N] + m_ref[1, :N]
    y = jnp.dot(hm, w_ref[...], preferred_element_type=jnp.float32,
                precision=lax.Precision.HIGHEST)
    o_ref[...] = jnp.maximum(y + b_ref[...], 0.0)


_layer_tc = pl.pallas_call(
    _layer_body,
    out_shape=jax.ShapeDtypeStruct((N, D), jnp.float32),
)


def _head_body(h_ref, wmu_ref, bmu_ref, wlv_ref, blv_ref,
               wd1_ref, bd1_ref, wd2_ref, bd2_ref, eps_ref,
               z_ref, kl_ref):
    h = h_ref[...]
    mm = functools.partial(jnp.dot, preferred_element_type=jnp.float32,
                           precision=lax.Precision.HIGHEST)
    mu = mm(h, wmu_ref[...]) + bmu_ref[...]
    lv = mm(h, wlv_ref[...]) + blv_ref[...]
    elv = jnp.exp(lv)
    kl_terms = 1.0 + 2.0 * lv - mu * mu - elv * elv
    kl = -0.5 * (jnp.sum(kl_terms) / jnp.float32(N))
    kl_ref[...] = jnp.full((8, 128), kl, dtype=jnp.float32)
    z_lat = eps_ref[...] * elv + mu
    z1 = jnp.maximum(mm(z_lat, wd1_ref[...]) + bd1_ref[...], 0.0)
    z_ref[...] = mm(z1, wd2_ref[...]) + bd2_ref[...]


_head_tc = pl.pallas_call(
    _head_body,
    out_shape=(
        jax.ShapeDtypeStruct((N, D), jnp.float32),
        jax.ShapeDtypeStruct((8, 128), jnp.float32),
    ),
)


def _loss_body(pv_ref, nv_ref, o_ref):
    n2 = jnp.float32(N) * jnp.float32(N)
    pos_weight = (n2 - 2.0) / 2.0
    norm = n2 / ((n2 - 2.0) * 2.0)
    pos_adj = jax.nn.sigmoid(pv_ref[...])
    pos_loss = -jnp.mean(jnp.log(pos_adj + EPS))
    neg_adj = jax.nn.sigmoid(nv_ref[...])
    neg_loss = -jnp.mean(jnp.log(1.0 - neg_adj + EPS))
    recon = norm * (pos_loss * pos_weight + neg_loss)
    o_ref[...] = jnp.full((8, 128), recon, dtype=jnp.float32)


_loss_tc = pl.pallas_call(
    _loss_body,
    out_shape=jax.ShapeDtypeStruct((8, 128), jnp.float32),
)


def kernel(x, edge_index, W0, b0, W1, b1, W2, b2, Wmu, bmu, Wlv, blv,
           Wd1, bd1, Wd2, bd2, eps, neg_edge_index):
    packed = (edge_index[0] | (edge_index[1] << 16)).reshape(NW, NCHS, CS)
    zeros = jnp.zeros((RPT, D), jnp.float32)

    h = x
    for W, b in ((W0, b0), (W1, b1), (W2, b2)):
        parts = _segsum(h, packed, zeros)
        h = _layer_tc(h, parts, W, b.reshape(1, D))

    z, klbuf = _head_tc(h, Wmu, bmu.reshape(1, D), Wlv, blv.reshape(1, D),
                        Wd1, bd1.reshape(1, D), Wd2, bd2.reshape(1, D), eps)

    eidx = jnp.stack([edge_index, neg_edge_index]).reshape(2, 2, NW, NCH, C)
    vals = _edge_dots(z, eidx)
    lossbuf = _loss_tc(vals[0].reshape(2500, 128), vals[1].reshape(2500, 128))
    return (lossbuf[0, 0], jnp.float32(0.0), klbuf[0, 0])
